# baseline (speedup 1.0000x reference)
import functools

import jax
import jax.numpy as jnp
from jax import lax
from jax.experimental import pallas as pl
from jax.experimental.pallas import tpu as pltpu
from jax.experimental.pallas import tpu_sc as plsc

IN_DIM = 8192
OUT_DIM = 8192
BATCH = 2048
LANES = 16
NUM_WORKERS = 32
J_PER_W = OUT_DIM // NUM_WORKERS
K = 8
N_CHUNKS = J_PER_W // K

_M0 = (0, 0, 0, 0, 0, 0, 0, 0, 1, 1, 1, 1, 1, 1, 1, 1)
_MA = (0, 0, 1, 1, 0, 0, 1, 1, -1, -1, 0, 0, -1, -1, 0, 0)
_MB = (0, 0, 0, 0, 1, 1, 1, 1, -1, -1, -1, -1, 0, 0, 0, 0)
_MAB = (0, 1, -1, 0, -1, 0, -2, -1, 1, 2, 0, 1, 0, 1, -1, 0)


def _coef_body(w_ref, m_ref, c_ref):
    w = w_ref[...]
    m = jnp.max(w, axis=-1, keepdims=True)
    e = jnp.exp(w - m)
    s = e / jnp.sum(e, axis=-1, keepdims=True)

    def fold(i):
        return jnp.sum(s * m_ref[i][None, :], axis=-1)

    c_ref[...] = jnp.stack(
        [fold(0), fold(1), fold(2), fold(3)], axis=0)


def _coefs(weights):
    masks = jnp.asarray((_M0, _MA, _MB, _MAB), jnp.float32)
    return pl.pallas_call(
        _coef_body,
        out_shape=jax.ShapeDtypeStruct((4, OUT_DIM), jnp.float32),
    )(weights, masks)


def _tr_body(x_ref, o_ref):
    o_ref[...] = x_ref[...].T


def _transpose(x, bm=256, bn=512):
    m, n = x.shape
    return pl.pallas_call(
        _tr_body,
        grid=(m // bm, n // bn),
        in_specs=[pl.BlockSpec((bm, bn), lambda i, j: (i, j))],
        out_specs=pl.BlockSpec((bn, bm), lambda i, j: (j, i)),
        out_shape=jax.ShapeDtypeStruct((n, m), x.dtype),
    )(x)


def _make_sc_body(batch):
    def _sc_body(xT_hbm, c_hbm, iab_hbm, out_hbm,
                 iab_v, c_v, ab_v, o_v,
                 sem_g, sem_o0, sem_o1):
        wid = lax.axis_index("s") * 2 + lax.axis_index("c")
        jbase = wid * J_PER_W
        pltpu.sync_copy(iab_hbm.at[pl.ds(2 * jbase, 2 * J_PER_W)], iab_v)
        for c in range(4):
            pltpu.sync_copy(
                c_hbm.at[c, pl.ds(jbase * LANES, J_PER_W * LANES)],
                c_v.at[c])

        sem_o = (sem_o0, sem_o1)

        def gather_descs(ci, slot):
            return (
                pltpu.make_async_copy(
                    xT_hbm.at[iab_v.at[pl.ds(ci * 2 * K, 2 * K)]],
                    ab_v.at[slot], sem_g),
            )

        def out_desc(ci, slot):
            return pltpu.make_async_copy(
                o_v.at[slot], out_hbm.at[pl.ds(jbase + ci * K, K)],
                sem_o[slot])

        for d in gather_descs(0, 0):
            d.start()

        def pair(i, carry):
            for s in (0, 1):
                ci = 2 * i + s
                for d in gather_descs(ci, s):
                    d.wait()

                @pl.when(ci + 1 < N_CHUNKS)
                def _():
                    for d in gather_descs(ci + 1, 1 - s):
                        d.start()

                @pl.when(ci >= 2)
                def _():
                    out_desc(ci - 2, s).wait()

                for jj in range(K):
                    jloc = ci * K + jj
                    c0 = c_v[0, pl.ds(jloc * LANES, LANES)]
                    ca = c_v[1, pl.ds(jloc * LANES, LANES)]
                    cb = c_v[2, pl.ds(jloc * LANES, LANES)]
                    cab = c_v[3, pl.ds(jloc * LANES, LANES)]

                    @plsc.parallel_loop(0, batch, step=LANES, unroll=8)
                    def _(off, s=s, jj=jj, c0=c0, ca=ca, cb=cb, cab=cab):
                        av = ab_v[s, 2 * jj, pl.ds(off, LANES)]
                        bv = ab_v[s, 2 * jj + 1, pl.ds(off, LANES)]
                        o_v[s, jj, pl.ds(off, LANES)] = (
                            (c0 + ca * av) + (cb + cab * av) * bv)

                out_desc(ci, s).start()
            return carry

        lax.fori_loop(0, N_CHUNKS // 2, pair, 0)
        out_desc(N_CHUNKS - 2, 0).wait()
        out_desc(N_CHUNKS - 1, 1).wait()

    return _sc_body


def _make_sc_call(batch):
    return functools.partial(
        pl.kernel,
        mesh=plsc.VectorSubcoreMesh(core_axis_name="c",
                                    subcore_axis_name="s"),
        out_type=jax.ShapeDtypeStruct((OUT_DIM, batch), jnp.float32),
        scratch_types=[
            pltpu.VMEM((2 * J_PER_W,), jnp.int32),
            pltpu.VMEM((4, J_PER_W * LANES), jnp.float32),
            pltpu.VMEM((2, 2 * K, batch), jnp.float32),
            pltpu.VMEM((2, K, batch), jnp.float32),
            pltpu.SemaphoreType.DMA,
            pltpu.SemaphoreType.DMA,
            pltpu.SemaphoreType.DMA,
        ],
    )(_make_sc_body(batch))


_HALVES = 1
_HB = BATCH // _HALVES
_sc_half = _make_sc_call(_HB)


@jax.jit
def kernel(x, weights, idx_a, idx_b):
    ia = idx_a.astype(jnp.int32)
    ib = idx_b.astype(jnp.int32)
    coefs = _coefs(weights)
    coefs_b = jnp.reshape(
        jnp.broadcast_to(coefs[:, :, None], (4, OUT_DIM, LANES)),
        (4, OUT_DIM * LANES))
    iab = jnp.reshape(jnp.stack([ia, ib], axis=1), (2 * OUT_DIM,))
    outs = []
    for h in range(_HALVES):
        xT_h = x[h * _HB:(h + 1) * _HB].T
        outs.append(_sc_half(xT_h, coefs_b, iab))
    return jnp.concatenate([o.T for o in outs], axis=0)


# speedup vs baseline: 1.0385x; 1.0385x over previous
import functools

import jax
import jax.numpy as jnp
from jax import lax
from jax.experimental import pallas as pl
from jax.experimental.pallas import tpu as pltpu
from jax.experimental.pallas import tpu_sc as plsc

IN_DIM = 8192
OUT_DIM = 8192
BATCH = 2048
LANES = 16
NUM_WORKERS = 32
J_PER_W = OUT_DIM // NUM_WORKERS
K = 8
N_CHUNKS = J_PER_W // K

_M0 = (0, 0, 0, 0, 0, 0, 0, 0, 1, 1, 1, 1, 1, 1, 1, 1)
_MA = (0, 0, 1, 1, 0, 0, 1, 1, -1, -1, 0, 0, -1, -1, 0, 0)
_MB = (0, 0, 0, 0, 1, 1, 1, 1, -1, -1, -1, -1, 0, 0, 0, 0)
_MAB = (0, 1, -1, 0, -1, 0, -2, -1, 1, 2, 0, 1, 0, 1, -1, 0)


def _coef_body(w_ref, m_ref, c_ref):
    w = w_ref[...]
    m = jnp.max(w, axis=-1, keepdims=True)
    e = jnp.exp(w - m)
    s = e / jnp.sum(e, axis=-1, keepdims=True)

    def fold(i):
        return jnp.sum(s * m_ref[i][None, :], axis=-1)

    c_ref[...] = jnp.stack(
        [fold(0), fold(1), fold(2), fold(3)], axis=0)


def _coefs(weights):
    masks = jnp.asarray((_M0, _MA, _MB, _MAB), jnp.float32)
    return pl.pallas_call(
        _coef_body,
        out_shape=jax.ShapeDtypeStruct((4, OUT_DIM), jnp.float32),
    )(weights, masks)


def _make_sc_body(batch):
    def _sc_body(xT_hbm, c_hbm, ia_hbm, ib_hbm, out_hbm,
                 ia_v, ib_v, c_v, a_v, b_v, o_v,
                 sem_a, sem_b, sem_o0, sem_o1):
        wid = lax.axis_index("s") * 2 + lax.axis_index("c")
        jbase = wid * J_PER_W
        pltpu.sync_copy(ia_hbm.at[pl.ds(jbase, J_PER_W)], ia_v)
        pltpu.sync_copy(ib_hbm.at[pl.ds(jbase, J_PER_W)], ib_v)
        for c in range(4):
            pltpu.sync_copy(
                c_hbm.at[c, pl.ds(jbase * LANES, J_PER_W * LANES)],
                c_v.at[c])

        sem_o = (sem_o0, sem_o1)

        def gather_descs(ci, slot):
            return (
                pltpu.make_async_copy(
                    xT_hbm.at[ia_v.at[pl.ds(ci * K, K)]], a_v.at[slot],
                    sem_a),
                pltpu.make_async_copy(
                    xT_hbm.at[ib_v.at[pl.ds(ci * K, K)]], b_v.at[slot],
                    sem_b),
            )

        def out_desc(ci, slot):
            return pltpu.make_async_copy(
                o_v.at[slot], out_hbm.at[pl.ds(jbase + ci * K, K)],
                sem_o[slot])

        for d in gather_descs(0, 0):
            d.start()

        def pair(i, carry):
            for s in (0, 1):
                ci = 2 * i + s
                for d in gather_descs(ci, s):
                    d.wait()

                @pl.when(ci + 1 < N_CHUNKS)
                def _():
                    for d in gather_descs(ci + 1, 1 - s):
                        d.start()

                @pl.when(ci >= 2)
                def _():
                    out_desc(ci - 2, s).wait()

                for jj in range(K):
                    jloc = ci * K + jj
                    c0 = c_v[0, pl.ds(jloc * LANES, LANES)]
                    ca = c_v[1, pl.ds(jloc * LANES, LANES)]
                    cb = c_v[2, pl.ds(jloc * LANES, LANES)]
                    cab = c_v[3, pl.ds(jloc * LANES, LANES)]

                    @plsc.parallel_loop(0, batch, step=LANES, unroll=8)
                    def _(off, s=s, jj=jj, c0=c0, ca=ca, cb=cb, cab=cab):
                        av = a_v[s, jj, pl.ds(off, LANES)]
                        bv = b_v[s, jj, pl.ds(off, LANES)]
                        o_v[s, jj, pl.ds(off, LANES)] = (
                            (c0 + ca * av) + (cb + cab * av) * bv)

                out_desc(ci, s).start()
            return carry

        lax.fori_loop(0, N_CHUNKS // 2, pair, 0)
        out_desc(N_CHUNKS - 2, 0).wait()
        out_desc(N_CHUNKS - 1, 1).wait()

    return _sc_body


def _make_sc_call(batch):
    return functools.partial(
        pl.kernel,
        mesh=plsc.VectorSubcoreMesh(core_axis_name="c",
                                    subcore_axis_name="s"),
        out_type=jax.ShapeDtypeStruct((OUT_DIM, batch), jnp.float32),
        scratch_types=[
            pltpu.VMEM((J_PER_W,), jnp.int32),
            pltpu.VMEM((J_PER_W,), jnp.int32),
            pltpu.VMEM((4, J_PER_W * LANES), jnp.float32),
            pltpu.VMEM((2, K, batch), jnp.float32),
            pltpu.VMEM((2, K, batch), jnp.float32),
            pltpu.VMEM((2, K, batch), jnp.float32),
            pltpu.SemaphoreType.DMA,
            pltpu.SemaphoreType.DMA,
            pltpu.SemaphoreType.DMA,
            pltpu.SemaphoreType.DMA,
        ],
    )(_make_sc_body(batch))


_sc_call = _make_sc_call(BATCH)


@jax.jit
def kernel(x, weights, idx_a, idx_b):
    ia = idx_a.astype(jnp.int32)
    ib = idx_b.astype(jnp.int32)
    coefs = _coefs(weights)
    coefs_b = jnp.reshape(
        jnp.broadcast_to(coefs[:, :, None], (4, OUT_DIM, LANES)),
        (4, OUT_DIM * LANES))
    xT = x.T
    outT = _sc_call(xT, coefs_b, ia, ib)
    return outT.T
